# baseline (speedup 1.0000x reference)
import jax
import jax.numpy as jnp
from jax.experimental import pallas as pl
from jax.experimental.pallas import tpu as pltpu
from jax.experimental.pallas import tpu_sc as plsc

S = 2048
D = 1024
E = 64
F = 512
T = 128
G = S // T + E
SUB = 8
RL = D // SUB
NSUB = S * SUB
SCW = 128


def _sc_scatter_rows(x, idx_sub, n_rows):
    xs = x.reshape(NSUB, RL)
    mesh = plsc.VectorSubcoreMesh(core_axis_name="c", subcore_axis_name="s")

    @pl.kernel(out_type=jax.ShapeDtypeStruct((n_rows * SUB, RL), x.dtype),
               mesh=mesh, scratch_types=[])
    def k(x_hbm, i_hbm, o_hbm):
        def body(x_vmem, i_vmem):
            pltpu.sync_copy(x_vmem, o_hbm.at[i_vmem.at[0]])

        pltpu.emit_pipeline(
            body,
            grid=(NSUB // SCW,),
            in_specs=[pl.BlockSpec((SCW, RL), index_map=lambda i: (i, 0)),
                      pl.BlockSpec((1, SCW), index_map=lambda i: (0, i))],
            out_specs=[],
            core_axis_name=("c", "s"),
            dimension_semantics=(pltpu.PARALLEL,),
        )(x_hbm, i_hbm)

    return k(xs, idx_sub).reshape(n_rows, D)


def _sc_gather_rows(data, idx_sub):
    ds = data.reshape(data.shape[0] * SUB, RL)
    mesh = plsc.VectorSubcoreMesh(core_axis_name="c", subcore_axis_name="s")

    @pl.kernel(out_type=jax.ShapeDtypeStruct((NSUB, RL), data.dtype),
               mesh=mesh, scratch_types=[])
    def k(d_hbm, i_hbm, o_hbm):
        def body(i_vmem, o_vmem):
            pltpu.sync_copy(d_hbm.at[i_vmem.at[0]], o_vmem)

        pltpu.emit_pipeline(
            body,
            grid=(NSUB // SCW,),
            in_specs=[pl.BlockSpec((1, SCW), index_map=lambda i: (0, i))],
            out_specs=[pl.BlockSpec((SCW, RL), index_map=lambda i: (i, 0))],
            core_axis_name=("c", "s"),
            dimension_semantics=(pltpu.PARALLEL,),
        )(i_hbm, o_hbm)

    return k(ds, idx_sub).reshape(S, D)


def _router_body(x_ref, gw_ref, dest_ref, te_ref):
    logits = jax.lax.dot_general(
        x_ref[...], gw_ref[...], (((1,), (1,)), ((), ())),
        preferred_element_type=jnp.float32)
    rw = jax.nn.softmax(logits, axis=-1)
    eid = jnp.argmax(rw, axis=-1, keepdims=True).astype(jnp.int32)

    cols = jax.lax.broadcasted_iota(jnp.int32, (S, E), 1)
    m = (cols == eid)
    m_f = m.astype(jnp.float32)

    r_io = jax.lax.broadcasted_iota(jnp.int32, (S, S), 0)
    c_io = jax.lax.broadcasted_iota(jnp.int32, (S, S), 1)
    tri = (c_io < r_io).astype(jnp.bfloat16)
    rankmat = jax.lax.dot_general(
        tri, m.astype(jnp.bfloat16), (((1,), (0,)), ((), ())),
        preferred_element_type=jnp.float32)
    rank_tok = jnp.sum(rankmat * m_f, axis=1, keepdims=True)

    counts = jnp.sum(m_f, axis=0, keepdims=True).astype(jnp.int32)
    num_tiles = (counts + (T - 1)) // T
    lt = (jax.lax.broadcasted_iota(jnp.int32, (E, E), 0)
          <= jax.lax.broadcasted_iota(jnp.int32, (E, E), 1)).astype(jnp.float32)
    tiles_cum = jax.lax.dot_general(
        num_tiles.astype(jnp.float32), lt, (((1,), (0,)), ((), ())),
        preferred_element_type=jnp.float32).astype(jnp.int32)
    tile_start = tiles_cum - num_tiles

    dest_base = jnp.sum(m_f * tile_start.astype(jnp.float32), axis=1,
                        keepdims=True)
    dest = (dest_base * T + rank_tok).astype(jnp.int32)
    sub = jax.lax.broadcasted_iota(jnp.int32, (S, SUB), 1)
    dest_ref[...] = dest * SUB + sub

    total = tiles_cum[:, E - 1:E]
    ti = jax.lax.broadcasted_iota(jnp.int32, (G, E), 0)
    te = jnp.sum((jnp.broadcast_to(tiles_cum, (G, E)) <= ti).astype(jnp.int32),
                 axis=1, keepdims=True)
    lanes = jax.lax.broadcasted_iota(jnp.int32, (1, E), 1)
    last_e = jnp.max(jnp.where(counts > 0, lanes, 0), axis=1, keepdims=True)
    ti_col = jax.lax.broadcasted_iota(jnp.int32, (G, 1), 0)
    te_ref[...] = jnp.where(ti_col < total, te, last_e)


def _mlp_body(te_ref, x_ref, guw_ref, dw_ref, o_ref):
    x = x_ref[...].astype(jnp.bfloat16)
    gu = jax.lax.dot_general(
        x, guw_ref[0].astype(jnp.bfloat16), (((1,), (1,)), ((), ())),
        preferred_element_type=jnp.float32)
    g = gu[:, :F]
    u = gu[:, F:]
    h = g * jax.lax.logistic(g) * u
    o_ref[...] = jax.lax.dot_general(
        h.astype(jnp.bfloat16), dw_ref[0].astype(jnp.bfloat16),
        (((1,), (1,)), ((), ())),
        preferred_element_type=jnp.float32)


def kernel(hidden_states, gate_W, gate_up_W, down_W):
    B, S_, D_ = hidden_states.shape
    x = hidden_states.reshape(S, D)

    dest2d, te2d = pl.pallas_call(
        _router_body,
        out_shape=(jax.ShapeDtypeStruct((S, SUB), jnp.int32),
                   jax.ShapeDtypeStruct((G, 1), jnp.int32)),
    )(x, gate_W)
    dest_sub = dest2d.reshape(1, NSUB)
    tile_expert = te2d[:, 0]

    xp = _sc_scatter_rows(x, dest_sub, G * T)

    grid_spec = pltpu.PrefetchScalarGridSpec(
        num_scalar_prefetch=1,
        grid=(G,),
        in_specs=[
            pl.BlockSpec((T, D), lambda i, te: (i, 0)),
            pl.BlockSpec((1, 2 * F, D), lambda i, te: (te[i], 0, 0)),
            pl.BlockSpec((1, D, F), lambda i, te: (te[i], 0, 0)),
        ],
        out_specs=pl.BlockSpec((T, D), lambda i, te: (i, 0)),
    )
    outp = pl.pallas_call(
        _mlp_body,
        grid_spec=grid_spec,
        out_shape=jax.ShapeDtypeStruct((G * T, D), jnp.float32),
    )(tile_expert, xp, gate_up_W, down_W)

    out = _sc_gather_rows(outp, dest_sub)
    return out.reshape(B, S_, D_)


# speedup vs baseline: 1.4507x; 1.4507x over previous
import jax
import jax.numpy as jnp
from jax.experimental import pallas as pl
from jax.experimental.pallas import tpu as pltpu
from jax.experimental.pallas import tpu_sc as plsc

S = 2048
D = 1024
E = 64
F = 512
T = 128
G = S // T + E
SCW = 32
IPAD = 128


def _sc_scatter_rows(x, idx_pad, n_rows):
    mesh = plsc.VectorSubcoreMesh(core_axis_name="c", subcore_axis_name="s")

    @pl.kernel(out_type=jax.ShapeDtypeStruct((n_rows, D), x.dtype),
               mesh=mesh, scratch_types=[])
    def k(x_hbm, i_hbm, o_hbm):
        def body(x_vmem, i_vmem):
            pltpu.sync_copy(x_vmem, o_hbm.at[i_vmem.at[0, pl.ds(0, SCW)]])

        pltpu.emit_pipeline(
            body,
            grid=(S // SCW,),
            in_specs=[pl.BlockSpec((SCW, D), index_map=lambda i: (i, 0)),
                      pl.BlockSpec((1, IPAD), index_map=lambda i: (i, 0))],
            out_specs=[],
            core_axis_name=("c", "s"),
            dimension_semantics=(pltpu.PARALLEL,),
        )(x_hbm, i_hbm)

    return k(x, idx_pad)


def _sc_gather_rows(data, idx_pad):
    mesh = plsc.VectorSubcoreMesh(core_axis_name="c", subcore_axis_name="s")

    @pl.kernel(out_type=jax.ShapeDtypeStruct((S, D), data.dtype),
               mesh=mesh, scratch_types=[])
    def k(d_hbm, i_hbm, o_hbm):
        def body(i_vmem, o_vmem):
            pltpu.sync_copy(d_hbm.at[i_vmem.at[0, pl.ds(0, SCW)]], o_vmem)

        pltpu.emit_pipeline(
            body,
            grid=(S // SCW,),
            in_specs=[pl.BlockSpec((1, IPAD), index_map=lambda i: (i, 0))],
            out_specs=[pl.BlockSpec((SCW, D), index_map=lambda i: (i, 0))],
            core_axis_name=("c", "s"),
            dimension_semantics=(pltpu.PARALLEL,),
        )(i_hbm, o_hbm)

    return k(data, idx_pad)


def _router_body(x_ref, gw_ref, dest_ref, te_ref):
    logits = jax.lax.dot_general(
        x_ref[...], gw_ref[...], (((1,), (1,)), ((), ())),
        preferred_element_type=jnp.float32)
    rw = jax.nn.softmax(logits, axis=-1)
    eid = jnp.argmax(rw, axis=-1, keepdims=True).astype(jnp.int32)

    cols = jax.lax.broadcasted_iota(jnp.int32, (S, E), 1)
    m = (cols == eid)
    m_f = m.astype(jnp.float32)

    r_io = jax.lax.broadcasted_iota(jnp.int32, (S, S), 0)
    c_io = jax.lax.broadcasted_iota(jnp.int32, (S, S), 1)
    tri = (c_io < r_io).astype(jnp.bfloat16)
    rankmat = jax.lax.dot_general(
        tri, m.astype(jnp.bfloat16), (((1,), (0,)), ((), ())),
        preferred_element_type=jnp.float32)
    rank_tok = jnp.sum(rankmat * m_f, axis=1, keepdims=True)

    counts = jnp.sum(m_f, axis=0, keepdims=True).astype(jnp.int32)
    num_tiles = (counts + (T - 1)) // T
    lt = (jax.lax.broadcasted_iota(jnp.int32, (E, E), 0)
          <= jax.lax.broadcasted_iota(jnp.int32, (E, E), 1)).astype(jnp.float32)
    tiles_cum = jax.lax.dot_general(
        num_tiles.astype(jnp.float32), lt, (((1,), (0,)), ((), ())),
        preferred_element_type=jnp.float32).astype(jnp.int32)
    tile_start = tiles_cum - num_tiles

    dest_base = jnp.sum(m_f * tile_start.astype(jnp.float32), axis=1,
                        keepdims=True)
    dest_ref[...] = (dest_base * T + rank_tok).astype(jnp.int32)

    total = tiles_cum[:, E - 1:E]
    ti = jax.lax.broadcasted_iota(jnp.int32, (G, E), 0)
    te = jnp.sum((jnp.broadcast_to(tiles_cum, (G, E)) <= ti).astype(jnp.int32),
                 axis=1, keepdims=True)
    lanes = jax.lax.broadcasted_iota(jnp.int32, (1, E), 1)
    last_e = jnp.max(jnp.where(counts > 0, lanes, 0), axis=1, keepdims=True)
    ti_col = jax.lax.broadcasted_iota(jnp.int32, (G, 1), 0)
    te_ref[...] = jnp.where(ti_col < total, te, last_e)


def _mlp_body(te_ref, x_ref, guw_ref, dw_ref, o_ref):
    x = x_ref[...].astype(jnp.bfloat16)
    gu = jax.lax.dot_general(
        x, guw_ref[0].astype(jnp.bfloat16), (((1,), (1,)), ((), ())),
        preferred_element_type=jnp.float32)
    g = gu[:, :F]
    u = gu[:, F:]
    h = g * jax.lax.logistic(g) * u
    o_ref[...] = jax.lax.dot_general(
        h.astype(jnp.bfloat16), dw_ref[0].astype(jnp.bfloat16),
        (((1,), (1,)), ((), ())),
        preferred_element_type=jnp.float32)


def kernel(hidden_states, gate_W, gate_up_W, down_W):
    B, S_, D_ = hidden_states.shape
    x = hidden_states.reshape(S, D)

    dest2d, te2d = pl.pallas_call(
        _router_body,
        out_shape=(jax.ShapeDtypeStruct((S, 1), jnp.int32),
                   jax.ShapeDtypeStruct((G, 1), jnp.int32)),
    )(x, gate_W)
    idx_pad = jnp.zeros((S // SCW, IPAD), jnp.int32).at[:, :SCW].set(
        dest2d.reshape(S // SCW, SCW))
    tile_expert = te2d[:, 0]

    xp = _sc_scatter_rows(x, idx_pad, G * T)

    grid_spec = pltpu.PrefetchScalarGridSpec(
        num_scalar_prefetch=1,
        grid=(G,),
        in_specs=[
            pl.BlockSpec((T, D), lambda i, te: (i, 0)),
            pl.BlockSpec((1, 2 * F, D), lambda i, te: (te[i], 0, 0)),
            pl.BlockSpec((1, D, F), lambda i, te: (te[i], 0, 0)),
        ],
        out_specs=pl.BlockSpec((T, D), lambda i, te: (i, 0)),
    )
    outp = pl.pallas_call(
        _mlp_body,
        grid_spec=grid_spec,
        out_shape=jax.ShapeDtypeStruct((G * T, D), jnp.float32),
    )(tile_expert, xp, gate_up_W, down_W)

    out = _sc_gather_rows(outp, idx_pad)
    return out.reshape(B, S_, D_)
